# baseline (speedup 1.0000x reference)
import functools

import jax
import jax.numpy as jnp
from jax import lax
from jax.experimental import pallas as pl
from jax.experimental.pallas import tpu as pltpu
from jax.experimental.pallas import tpu_sc as plsc

_N = 10000
_E = 320000
_D = 64
_CH = 80
_G = _CH // 16
_NW = 32
_EPW = _E // _NW
_NCH = _EPW // _CH
_RPT = 624
_REM = _N - 16 * _RPT

_mesh = plsc.VectorSubcoreMesh(core_axis_name="c", subcore_axis_name="s")


@functools.partial(
    pl.kernel,
    mesh=_mesh,
    compiler_params=pltpu.CompilerParams(use_tc_tiling_on_sc=False),
    out_type=jax.ShapeDtypeStruct((2, _N, _D), jnp.float32),
    scratch_types=(
        [pltpu.VMEM((_EPW,), jnp.float32)]
        + [pltpu.VMEM((_CH,), jnp.int32)] * 2
        + [pltpu.VMEM((_CH,), jnp.int32)] * 2
        + [pltpu.VMEM((_CH, _D), jnp.float32)] * 2
        + [pltpu.VMEM_SHARED((_N, _D), jnp.float32)]
        + [pltpu.VMEM_SHARED((_N, _D), jnp.float32)]
        + [pltpu.SemaphoreType.DMA] * 6
    ),
)
def _edge_pass(t_hbm, src_hbm, dst_hbm, w_hbm, out_hbm,
               w_v, sidx0, sidx1, didx0, didx1, rows0, rows1, tab, acc,
               isem0, isem1, gsem0, gsem1, ssem0, ssem1):
  c = lax.axis_index("c")
  s = lax.axis_index("s")
  wid = s * 2 + c
  e0 = wid * _EPW

  sidx = (sidx0, sidx1)
  didx = (didx0, didx1)
  rows = (rows0, rows1)
  isem = (isem0, isem1)
  gsem = (gsem0, gsem1)
  ssem = (ssem0, ssem1)

  pltpu.sync_copy(t_hbm.at[pl.ds(s * _RPT, _RPT)], tab.at[pl.ds(s * _RPT, _RPT)])
  zv = jnp.zeros((16,), jnp.float32)

  def zrow(r, carry):
    for j in range(_D // 16):
      rows0[r, pl.ds(j * 16, 16)] = zv
    return carry

  lax.fori_loop(0, _CH, zrow, 0)
  for t in range(_RPT // _CH):
    pltpu.sync_copy(rows0, acc.at[pl.ds(s * _RPT + t * _CH, _CH)])
  pltpu.sync_copy(rows0.at[pl.ds(0, _RPT % _CH)],
                  acc.at[pl.ds(s * _RPT + (_RPT // _CH) * _CH, _RPT % _CH)])

  @pl.when(s == 0)
  def _():
    pltpu.sync_copy(t_hbm.at[pl.ds(16 * _RPT, _REM)],
                    tab.at[pl.ds(16 * _RPT, _REM)])
    pltpu.sync_copy(rows0.at[pl.ds(0, _REM)], acc.at[pl.ds(16 * _RPT, _REM)])

  pltpu.sync_copy(w_hbm.at[pl.ds(e0, _EPW)], w_v)
  plsc.subcore_barrier()

  def idx_start(ci, b):
    off = e0 + ci * _CH
    pltpu.async_copy(src_hbm.at[pl.ds(off, _CH)], sidx[b], isem[b])
    pltpu.async_copy(dst_hbm.at[pl.ds(off, _CH)], didx[b], isem[b])

  def idx_wait(ci, b):
    off = e0 + ci * _CH
    pltpu.make_async_copy(src_hbm.at[pl.ds(off, _CH)], sidx[b], isem[b]).wait()
    pltpu.make_async_copy(dst_hbm.at[pl.ds(off, _CH)], didx[b], isem[b]).wait()

  def gather_start(b):
    pltpu.async_copy(tab.at[sidx[b]], rows[b], gsem[b])

  def gather_wait(b):
    pltpu.make_async_copy(tab.at[sidx[b]], rows[b], gsem[b]).wait()

  def scat_start(b):
    pltpu.async_copy(rows[b], acc.at[didx[b]], ssem[b], add=True)

  def scat_wait(b):
    pltpu.make_async_copy(rows[b], acc.at[didx[b]], ssem[b]).wait()

  def scale(ci, b):
    rv = rows[b]
    for g in range(_G):
      wv16 = w_v[pl.ds(ci * _CH + g * 16, 16)]
      for l in range(16):
        wl = jnp.broadcast_to(wv16[l], (16,))
        r = g * 16 + l
        for j in range(_D // 16):
          sl = pl.ds(j * 16, 16)
          rv[r, sl] = rv[r, sl] * wl

  idx_start(0, 0)
  idx_wait(0, 0)
  gather_start(0)

  def pair(ip, carry):
    for b in range(2):
      ci = 2 * ip + b
      nb = 1 - b

      @pl.when(ci > 0)
      def _():
        scat_wait(nb)

      idx_start(ci + 1, nb)
      gather_wait(b)
      idx_wait(ci + 1, nb)
      gather_start(nb)
      scale(ci, b)
      scat_start(b)
    return carry

  lax.fori_loop(0, (_NCH - 1) // 2, pair, 0)

  scat_wait(1)
  gather_wait(0)
  scale(_NCH - 1, 0)
  scat_start(0)
  scat_wait(0)

  plsc.subcore_barrier()
  pltpu.sync_copy(acc.at[pl.ds(s * _RPT, _RPT)],
                  out_hbm.at[c, pl.ds(s * _RPT, _RPT)])

  @pl.when(s == 0)
  def _():
    pltpu.sync_copy(acc.at[pl.ds(16 * _RPT, _REM)],
                    out_hbm.at[c, pl.ds(16 * _RPT, _REM)])


_BR = 400


def _elu(x):
  return jnp.where(x > 0, x, jnp.exp(x) - 1.0)


def _mm_halves(x, w):
  n, k = x.shape

  def body(x_ref, w_ref, oa_ref, ob_ref):
    h = jnp.dot(x_ref[...], w_ref[...], preferred_element_type=jnp.float32)
    oa_ref[...] = h[:, :_D]
    ob_ref[...] = h[:, _D:]

  return pl.pallas_call(
      body,
      grid=(n // _BR,),
      in_specs=[pl.BlockSpec((_BR, k), lambda i: (i, 0)),
                pl.BlockSpec((k, 2 * _D), lambda i: (0, 0))],
      out_specs=[pl.BlockSpec((_BR, _D), lambda i: (i, 0)),
                 pl.BlockSpec((_BR, _D), lambda i: (i, 0))],
      out_shape=[jax.ShapeDtypeStruct((n, _D), jnp.float32),
                 jax.ShapeDtypeStruct((n, _D), jnp.float32)],
  )(x, w)


def _combine_mm(pa, pb, w):
  _, n, _ = pa.shape
  m = w.shape[1]

  def body(pa_ref, pb_ref, w_ref, o_ref):
    za = _elu(pa_ref[0] + pa_ref[1])
    zb = _elu(pb_ref[0] + pb_ref[1])
    o_ref[...] = (jnp.dot(za, w_ref[:_D], preferred_element_type=jnp.float32)
                  + jnp.dot(zb, w_ref[_D:], preferred_element_type=jnp.float32))

  return pl.pallas_call(
      body,
      grid=(n // _BR,),
      in_specs=[pl.BlockSpec((2, _BR, _D), lambda i: (0, i, 0)),
                pl.BlockSpec((2, _BR, _D), lambda i: (0, i, 0)),
                pl.BlockSpec((2 * _D, m), lambda i: (0, 0))],
      out_specs=pl.BlockSpec((_BR, m), lambda i: (i, 0)),
      out_shape=jax.ShapeDtypeStruct((n, m), jnp.float32),
  )(pa, pb, w)


def _combine_elu(p):
  _, n, k = p.shape

  def body(p_ref, o_ref):
    o_ref[...] = _elu(p_ref[0] + p_ref[1])

  return pl.pallas_call(
      body,
      grid=(n // _BR,),
      in_specs=[pl.BlockSpec((2, _BR, k), lambda i: (0, i, 0))],
      out_specs=pl.BlockSpec((_BR, k), lambda i: (i, 0)),
      out_shape=jax.ShapeDtypeStruct((n, k), jnp.float32),
  )(p)


def kernel(X_o, edge_index, edge_weight, W1, W2):
  src_r = edge_index[0]
  dst_r = edge_index[1]
  w_r = edge_weight

  ha, hb = _mm_halves(X_o, W1)
  pa = _edge_pass(ha, src_r, dst_r, w_r)
  pb = _edge_pass(hb, src_r, dst_r, w_r)
  h2 = _combine_mm(pa, pb, W2)
  p2 = _edge_pass(h2, src_r, dst_r, w_r)
  return _combine_elu(p2)
